# baseline (speedup 1.0000x reference)
import jax
import jax.numpy as jnp
from jax import lax
from jax.experimental import pallas as pl
from jax.experimental.pallas import tpu as pltpu

B = 16384
D = 128
BB = 2048
CHUNKS = BB // D


def _body(pk_ref, v1_ref, v2_ref, out_ref):
    i = pl.program_id(0)

    ut = pk_ref[0, :, :]
    d2t = pk_ref[2, :, :]
    s1 = jnp.max(ut)
    s2 = jnp.max(ut - d2t)

    p2m = jnp.minimum(d2t, s1)
    p1m = jnp.minimum(pk_ref[1, :, :], s2)

    a_idx = lax.broadcasted_iota(jnp.int32, (D, BB), 0)
    c_idx = lax.broadcasted_iota(jnp.int32, (D, BB), 1) // D
    onehot = jnp.where(a_idx == c_idx + i * CHUNKS, 1.0, 0.0).astype(jnp.float32)

    dims = (((1,), (0,)), ((), ()))
    w2 = lax.dot_general(p2m, onehot, dims, preferred_element_type=jnp.float32)
    w1 = lax.dot_general(p1m, onehot, dims, preferred_element_type=jnp.float32)

    for k in range(CHUNKS):
        rows = pl.ds(k * D, D)
        cols = slice(k * D, (k + 1) * D)
        out_ref[rows, :] = v2_ref[rows, :] * w2[:, cols] + v1_ref[rows, :] * w1[:, cols]


def kernel(u, d1, d2, v1, v2):
    n_blocks = B // BB
    pk = jnp.stack(
        [
            u.reshape(B // D, D).T,
            d1.reshape(B // D, D).T,
            d2.reshape(B // D, D).T,
        ]
    )
    pkspec = pl.BlockSpec((3, B // D, D), lambda i: (0, 0, 0))
    big = pl.BlockSpec((BB, D), lambda i: (i, 0))
    return pl.pallas_call(
        _body,
        grid=(n_blocks,),
        in_specs=[pkspec, big, big],
        out_specs=big,
        out_shape=jax.ShapeDtypeStruct((B, D), v1.dtype),
        compiler_params=pltpu.CompilerParams(
            dimension_semantics=(pltpu.PARALLEL,),
            allow_input_fusion=[True, False, False],
        ),
    )(pk, v1, v2)


# speedup vs baseline: 1.2686x; 1.2686x over previous
import jax
import jax.numpy as jnp
from jax import lax
from jax.experimental import pallas as pl
from jax.experimental.pallas import tpu as pltpu

B = 16384
D = 128
BB = 8192
CHUNKS = BB // D


def _body(pk_ref, v1_ref, v2_ref, out_ref):
    i = pl.program_id(0)

    ut = pk_ref[0, :, :]
    d2t = pk_ref[2, :, :]
    s1 = jnp.max(ut)
    s2 = jnp.max(ut - d2t)

    p2m = jnp.minimum(d2t, s1)
    p1m = jnp.minimum(pk_ref[1, :, :], s2)

    a_idx = lax.broadcasted_iota(jnp.int32, (D, BB), 0)
    c_idx = lax.broadcasted_iota(jnp.int32, (D, BB), 1) // D
    onehot = jnp.where(a_idx == c_idx + i * CHUNKS, 1.0, 0.0).astype(jnp.float32)

    dims = (((1,), (0,)), ((), ()))
    w2 = lax.dot_general(p2m, onehot, dims, preferred_element_type=jnp.float32)
    w1 = lax.dot_general(p1m, onehot, dims, preferred_element_type=jnp.float32)

    for k in range(CHUNKS):
        rows = pl.ds(k * D, D)
        cols = slice(k * D, (k + 1) * D)
        out_ref[rows, :] = v2_ref[rows, :] * w2[:, cols] + v1_ref[rows, :] * w1[:, cols]


def kernel(u, d1, d2, v1, v2):
    n_blocks = B // BB
    pk = jnp.stack(
        [
            u.reshape(B // D, D).T,
            d1.reshape(B // D, D).T,
            d2.reshape(B // D, D).T,
        ]
    )
    pkspec = pl.BlockSpec((3, B // D, D), lambda i: (0, 0, 0))
    big = pl.BlockSpec((BB, D), lambda i: (i, 0))
    return pl.pallas_call(
        _body,
        grid=(n_blocks,),
        in_specs=[pkspec, big, big],
        out_specs=big,
        out_shape=jax.ShapeDtypeStruct((B, D), v1.dtype),
        compiler_params=pltpu.CompilerParams(
            dimension_semantics=(pltpu.PARALLEL,),
            allow_input_fusion=[True, False, False],
        ),
    )(pk, v1, v2)
